# baseline (speedup 1.0000x reference)
import functools

import jax
import jax.numpy as jnp
from jax import lax
from jax.experimental import pallas as pl
from jax.experimental.pallas import tpu as pltpu
from jax.experimental.pallas import tpu_sc as plsc

_B = 4096
_NUM = 13
_NCAT = 26
_VOCAB = 100000
_H = 64

_NC = 2
_NS = 16
_NW = _NC * _NS
_PER_W = _B * _NCAT // _NW
_CHUNK = 128
_NCHUNK = _PER_W // _CHUNK

_BB = 512


def _mlp_body(x_ref, w1_ref, b1_ref, w2_ref, b2_ref, out_ref):
    x = x_ref[...]
    for n in range(_NUM):
        h = x[:, n:n + 1] * w1_ref[n:n + 1, :] + b1_ref[n:n + 1, :]
        h = 0.5 * h * (1.0 + lax.erf(h * 0.7071067811865476))
        t = jnp.dot(h, w2_ref[n], preferred_element_type=jnp.float32)
        out_ref[:, n, :] = t + b2_ref[n:n + 1, :]


def _num_tokens(x_num, W1, b1, W2, b2):
    return pl.pallas_call(
        _mlp_body,
        grid=(_B // _BB,),
        in_specs=[
            pl.BlockSpec((_BB, _NUM), lambda i: (i, 0)),
            pl.BlockSpec((_NUM, _H), lambda i: (0, 0)),
            pl.BlockSpec((_NUM, _H), lambda i: (0, 0)),
            pl.BlockSpec((_NUM, _H, _H), lambda i: (0, 0, 0)),
            pl.BlockSpec((_NUM, _H), lambda i: (0, 0)),
        ],
        out_specs=pl.BlockSpec((_BB, _NUM, _H), lambda i: (i, 0, 0)),
        out_shape=jax.ShapeDtypeStruct((_B, _NUM, _H), jnp.float32),
    )(x_num, W1, b1, W2, b2)


def _sc_gather_body(table_hbm, idx_hbm, out_hbm, idx_v, rows_v, sem):
    wid = lax.axis_index("s") * _NC + lax.axis_index("c")
    base = pl.multiple_of(wid * _PER_W, _PER_W)
    pltpu.sync_copy(idx_hbm.at[pl.ds(base, _PER_W)], idx_v)

    def body(j, carry):
        off = pl.multiple_of(j * _CHUNK, _CHUNK)
        pltpu.async_copy(table_hbm.at[idx_v.at[pl.ds(off, _CHUNK)]], rows_v, sem).wait()
        pltpu.sync_copy(rows_v, out_hbm.at[pl.ds(base + off, _CHUNK)])
        return carry

    lax.fori_loop(0, _NCHUNK, body, 0)


def _sc_gather(tables_flat, flat_idx):
    mesh = plsc.VectorSubcoreMesh(core_axis_name="c", subcore_axis_name="s")
    run = functools.partial(
        pl.kernel,
        out_type=jax.ShapeDtypeStruct((_B * _NCAT, _H), jnp.float32),
        mesh=mesh,
        scratch_types=[
            pltpu.VMEM((_PER_W,), jnp.int32),
            pltpu.VMEM((_CHUNK, _H), jnp.float32),
            pltpu.SemaphoreType.DMA,
        ],
        compiler_params=pltpu.CompilerParams(use_tc_tiling_on_sc=False),
    )(_sc_gather_body)
    return run(tables_flat, flat_idx)


def kernel(x_num, x_cat, W1, b1, W2, b2, tables, cls_token):
    batch = x_num.shape[0]
    tables_flat = tables.reshape(_NCAT * _VOCAB, _H)
    flat_idx = (x_cat + (jnp.arange(_NCAT, dtype=jnp.int32) * _VOCAB)[None, :]).reshape(-1)
    cat_tokens = _sc_gather(tables_flat, flat_idx).reshape(batch, _NCAT, _H)
    num_tokens = _num_tokens(x_num, W1, b1, W2, b2)
    cls = jnp.broadcast_to(cls_token, (batch, 1, _H))
    return jnp.concatenate([cls, num_tokens, cat_tokens], axis=1)
